# baseline (speedup 1.0000x reference)
import functools

import jax
import jax.numpy as jnp
from jax import lax
from jax.experimental import pallas as pl
from jax.experimental.pallas import tpu as pltpu
from jax.experimental.pallas import tpu_sc as plsc

_OFFSET = 100
_LANES = 16
_GW = 128
_GPC = 4
_NBUF = 2


def kernel(x, table):
    B0, S = x.shape
    D = table.shape[1]
    B = B0 * S
    n_lines = B * D // _GW
    bblocks = B0 // _GW
    dts = D // 8

    info = plsc.get_sparse_core_info()
    nw = info.num_cores * info.num_subcores
    n_groups = B // _GW
    g_per_w = n_groups // nw
    assert n_groups % nw == 0 and g_per_w % (_GPC * _NBUF) == 0
    n_chunks = g_per_w // _GPC

    xt2 = jnp.transpose(x).reshape(n_groups, _GW)
    mesh = plsc.VectorSubcoreMesh(core_axis_name="c", subcore_axis_name="s")

    @functools.partial(
        pl.kernel,
        mesh=mesh,
        out_type=jax.ShapeDtypeStruct((n_lines, _GW), jnp.float32),
        scratch_types=[
            pltpu.VMEM((_NBUF, _GPC, _GW), jnp.int32),
            pltpu.VMEM((_NBUF, _GPC * _GW, D), jnp.float32),
            pltpu.VMEM((_NBUF, _GPC, D, _GW), jnp.float32),
            pltpu.SemaphoreType.DMA,
            pltpu.SemaphoreType.DMA,
            pltpu.SemaphoreType.DMA,
            pltpu.SemaphoreType.DMA,
            pltpu.SemaphoreType.DMA,
            pltpu.SemaphoreType.DMA,
        ],
        compiler_params=pltpu.CompilerParams(
            use_tc_tiling_on_sc=False, needs_layout_passes=False),
    )
    def emb(idx_hbm, table_hbm, out_hbm, idx_v, rows_v, t_v,
            si0, si1, sg0, sg1, so0, so1):
        wid = lax.axis_index("s") * info.num_cores + lax.axis_index("c")
        g_base = wid * g_per_w
        si = (si0, si1)
        sg = (sg0, sg1)
        so = (so0, so1)
        biota = lax.iota(jnp.int32, _LANES)

        def idx_copy(ci, b):
            return pltpu.make_async_copy(
                idx_hbm.at[pl.ds(g_base + ci * _GPC, _GPC)], idx_v.at[b], si[b])

        def out_copies(ci, b):
            copies = []
            for r in range(_GPC):
                g = g_base + ci * _GPC + r
                s = g // bblocks
                bb = g % bblocks
                line0 = s * (dts * 8 * bblocks) + bb * 8
                for dt in range(dts):
                    copies.append(pltpu.make_async_copy(
                        t_v.at[b].at[r].at[pl.ds(dt * 8, 8)],
                        out_hbm.at[pl.ds(line0 + dt * 8 * bblocks, 8)],
                        so[b]))
            return copies

        idx_copy(0, 0).start()
        idx_copy(1, 1).start()

        def chunk_pair(p, carry):
            for b in range(_NBUF):
                ci = p * _NBUF + b

                @pl.when(p > 0)
                def _():
                    for c in out_copies(ci - _NBUF, b):
                        c.wait()

                idx_copy(ci, b).wait()

                gathers = []
                for r in range(_GPC):
                    for l in range(_GW // _LANES):
                        sl = pl.ds(l * _LANES, _LANES)
                        idx_v[b, r, sl] = idx_v[b, r, sl] + _OFFSET
                    g = pltpu.make_async_copy(
                        table_hbm.at[idx_v.at[b].at[r]],
                        rows_v.at[b].at[pl.ds(r * _GW, _GW)],
                        sg[b])
                    g.start()
                    gathers.append(g)
                for g in gathers:
                    g.wait()

                out_copy_list = out_copies(ci, b)
                for r in range(_GPC):
                    grp = rows_v.at[b].at[pl.ds(r * _GW, _GW)]
                    for cg in range(_GW // _LANES):
                        bidx = biota + cg * _LANES
                        for d in range(D):
                            t_v[b, r, d, pl.ds(cg * _LANES, _LANES)] = (
                                plsc.load_gather(
                                    grp, [bidx, jnp.full((_LANES,), d, jnp.int32)]))
                    for dt in range(dts):
                        out_copy_list[r * dts + dt].start()

                @pl.when(p < n_chunks // _NBUF - 1)
                def _():
                    idx_copy(ci + _NBUF, b).start()
            return carry

        lax.fori_loop(0, n_chunks // _NBUF, chunk_pair, 0)
        for c in out_copies(n_chunks - 2, 0):
            c.wait()
        for c in out_copies(n_chunks - 1, 1):
            c.wait()

    out2 = emb(xt2, table)
    out = (out2.reshape(S, dts, bblocks, 8, _GW)
           .transpose(0, 1, 3, 2, 4)
           .reshape(S, D, B0)
           .transpose(2, 0, 1))
    return out


# speedup vs baseline: 6.8848x; 6.8848x over previous
import functools

import jax
import jax.numpy as jnp
from jax import lax
from jax.experimental import pallas as pl
from jax.experimental.pallas import tpu as pltpu
from jax.experimental.pallas import tpu_sc as plsc

_OFFSET = 100
_LANES = 16
_GW = 128
_GPC = 4
_NBUF = 2


def kernel(x, table):
    B0, S = x.shape
    D = table.shape[1]
    B = B0 * S
    n_lines = B * D // _GW
    bblocks = B0 // _GW
    dts = D // 8

    info = plsc.get_sparse_core_info()
    nw = info.num_cores * info.num_subcores
    n_groups = B // _GW
    g_per_w = n_groups // nw
    assert n_groups % nw == 0 and g_per_w % (_GPC * _NBUF) == 0
    n_chunks = g_per_w // _GPC

    xt2 = jnp.transpose(x).reshape(n_groups, _GW)
    mesh = plsc.VectorSubcoreMesh(core_axis_name="c", subcore_axis_name="s")

    @functools.partial(
        pl.kernel,
        mesh=mesh,
        out_type=jax.ShapeDtypeStruct((n_lines, _GW), jnp.float32),
        scratch_types=[
            pltpu.VMEM((_NBUF, _GPC, _GW), jnp.int32),
            pltpu.VMEM((_NBUF, _GPC * _GW, D), jnp.float32),
            pltpu.VMEM((_NBUF, _GPC, D, _GW), jnp.float32),
            pltpu.SemaphoreType.DMA,
            pltpu.SemaphoreType.DMA,
            pltpu.SemaphoreType.DMA,
            pltpu.SemaphoreType.DMA,
            pltpu.SemaphoreType.DMA,
            pltpu.SemaphoreType.DMA,
        ],
        compiler_params=pltpu.CompilerParams(
            use_tc_tiling_on_sc=False, needs_layout_passes=False),
    )
    def emb(idx_hbm, table_hbm, out_hbm, idx_v, rows_v, t_v,
            si0, si1, sg0, sg1, so0, so1):
        wid = lax.axis_index("s") * info.num_cores + lax.axis_index("c")
        g_base = wid * g_per_w
        si = (si0, si1)
        sg = (sg0, sg1)
        so = (so0, so1)
        biota = lax.iota(jnp.int32, _LANES)

        def idx_copy(ci, b):
            return pltpu.make_async_copy(
                idx_hbm.at[pl.ds(g_base + ci * _GPC, _GPC)], idx_v.at[b], si[b])

        def out_copies(ci, b):
            copies = []
            for r in range(_GPC):
                g = g_base + ci * _GPC + r
                s = g // bblocks
                bb = g % bblocks
                line0 = s * (dts * 8 * bblocks) + bb * 8
                for dt in range(dts):
                    copies.append(pltpu.make_async_copy(
                        t_v.at[b].at[r].at[pl.ds(dt * 8, 8)],
                        out_hbm.at[pl.ds(line0 + dt * 8 * bblocks, 8)],
                        so[b]))
            return copies

        idx_copy(0, 0).start()
        idx_copy(1, 1).start()

        def chunk_pair(p, carry):
            for b in range(_NBUF):
                ci = p * _NBUF + b

                @pl.when(p > 0)
                def _():
                    for c in out_copies(ci - _NBUF, b):
                        c.wait()

                idx_copy(ci, b).wait()

                gathers = []
                for r in range(_GPC):
                    for l in range(_GW // _LANES):
                        sl = pl.ds(l * _LANES, _LANES)
                        idx_v[b, r, sl] = idx_v[b, r, sl] + _OFFSET
                    g = pltpu.make_async_copy(
                        table_hbm.at[idx_v.at[b].at[r]],
                        rows_v.at[b].at[pl.ds(r * _GW, _GW)],
                        sg[b])
                    g.start()
                    gathers.append(g)
                for g in gathers:
                    g.wait()

                out_copy_list = out_copies(ci, b)
                for r in range(_GPC):
                    tg = t_v.at[b].at[r]

                    @functools.partial(
                        plsc.parallel_loop, 0, _GW, unroll=8)
                    def _(brow, _r=r):
                        bsp = jnp.full((_LANES,), 0, jnp.int32) + brow
                        for c in range(D // _LANES):
                            v = rows_v[b, _r * _GW + brow,
                                       pl.ds(c * _LANES, _LANES)]
                            plsc.store_scatter(
                                tg, [biota + c * _LANES, bsp], v)

                    for dt in range(dts):
                        out_copy_list[r * dts + dt].start()

                @pl.when(p < n_chunks // _NBUF - 1)
                def _():
                    idx_copy(ci + _NBUF, b).start()
            return carry

        lax.fori_loop(0, n_chunks // _NBUF, chunk_pair, 0)
        for c in out_copies(n_chunks - 2, 0):
            c.wait()
        for c in out_copies(n_chunks - 1, 1):
            c.wait()

    out2 = emb(xt2, table)
    out = (out2.reshape(S, dts, bblocks, 8, _GW)
           .transpose(0, 1, 3, 2, 4)
           .reshape(S, D, B0)
           .transpose(2, 0, 1))
    return out


# speedup vs baseline: 8.8109x; 1.2798x over previous
import functools

import jax
import jax.numpy as jnp
from jax import lax
from jax.experimental import pallas as pl
from jax.experimental.pallas import tpu as pltpu
from jax.experimental.pallas import tpu_sc as plsc

_OFFSET = 100
_LANES = 16
_GW = 128
_GPC = 4
_NBUF = 2
_NIDX = 4


def kernel(x, table):
    B0, S = x.shape
    D = table.shape[1]
    B = B0 * S
    n_lines = B * D // _GW
    bblocks = B0 // _GW
    dts = D // 8
    n_groups = B // _GW

    info = plsc.get_sparse_core_info()
    nw = info.num_cores * info.num_subcores
    g_per_w = n_groups // nw
    assert n_groups % nw == 0 and g_per_w % (_GPC * _NBUF) == 0
    n_chunks = g_per_w // _GPC

    xt2 = (jnp.transpose(x)
           .reshape(S // 8, 8, bblocks, _GW)
           .transpose(0, 2, 1, 3)
           .reshape(n_groups, _GW))
    mesh = plsc.VectorSubcoreMesh(core_axis_name="c", subcore_axis_name="s")

    @functools.partial(
        pl.kernel,
        mesh=mesh,
        out_type=jax.ShapeDtypeStruct((n_lines, _GW), jnp.float32),
        scratch_types=[
            pltpu.VMEM((_NIDX, _GPC, _GW), jnp.int32),
            pltpu.VMEM((_NBUF, _GPC * _GW, D), jnp.float32),
            pltpu.VMEM((_NBUF, _GPC, D, _GW), jnp.float32),
            pltpu.SemaphoreType.DMA,
            pltpu.SemaphoreType.DMA,
            pltpu.SemaphoreType.DMA,
            pltpu.SemaphoreType.DMA,
            pltpu.SemaphoreType.DMA,
            pltpu.SemaphoreType.DMA,
            pltpu.SemaphoreType.DMA,
            pltpu.SemaphoreType.DMA,
        ],
        compiler_params=pltpu.CompilerParams(
            use_tc_tiling_on_sc=False, needs_layout_passes=False),
    )
    def emb(idx_hbm, table_hbm, out_hbm, idx_v, rows_v, t_v,
            si0, si1, si2, si3, sg0, sg1, so0, so1):
        wid = lax.axis_index("s") * info.num_cores + lax.axis_index("c")
        g_base = wid * g_per_w
        si = (si0, si1, si2, si3)
        sg = (sg0, sg1)
        so = (so0, so1)
        biota = lax.iota(jnp.int32, _LANES)

        def idx_copy(ci, slot):
            return pltpu.make_async_copy(
                idx_hbm.at[pl.ds(g_base + ci * _GPC, _GPC)],
                idx_v.at[slot], si[slot])

        def line0_of(ci, r):
            l = g_base + ci * _GPC + r
            si_ = l // (bblocks * 8)
            rem = l % (bblocks * 8)
            bb = rem // 8
            sr = rem % 8
            s = si_ * 8 + sr
            return s * (dts * 8 * bblocks) + bb * 8

        def out_copies(ci, b):
            copies = []
            for r in range(_GPC):
                line0 = line0_of(ci, r)
                for dt in range(dts):
                    copies.append(pltpu.make_async_copy(
                        t_v.at[b].at[r].at[pl.ds(dt * 8, 8)],
                        out_hbm.at[pl.ds(line0 + dt * 8 * bblocks, 8)],
                        so[b]))
            return copies

        def fire_gathers(ci, slot, b):
            for r in range(_GPC):
                for l in range(_GW // _LANES):
                    sl = pl.ds(l * _LANES, _LANES)
                    idx_v[slot, r, sl] = idx_v[slot, r, sl] + _OFFSET
                pltpu.make_async_copy(
                    table_hbm.at[idx_v.at[slot].at[r]],
                    rows_v.at[b].at[pl.ds(r * _GW, _GW)],
                    sg[b]).start()

        def wait_gathers(slot, b):
            for r in range(_GPC):
                pltpu.make_async_copy(
                    table_hbm.at[idx_v.at[slot].at[r]],
                    rows_v.at[b].at[pl.ds(r * _GW, _GW)],
                    sg[b]).wait()

        def transpose_and_write(ci, slot, b):
            wait_gathers(slot, b)
            out_copy_list = out_copies(ci, b)
            for r in range(_GPC):
                tg = t_v.at[b].at[r]

                @functools.partial(plsc.parallel_loop, 0, _GW, unroll=8)
                def _(brow, _r=r):
                    bsp = jnp.full((_LANES,), 0, jnp.int32) + brow
                    for c in range(D // _LANES):
                        v = rows_v[b, _r * _GW + brow,
                                   pl.ds(c * _LANES, _LANES)]
                        plsc.store_scatter(
                            tg, [biota + c * _LANES, bsp], v)
                for dt in range(dts):
                    out_copy_list[r * dts + dt].start()

        def step(i, j):
            idx_copy(i, j % _NIDX).wait()
            static = isinstance(i, int)

            def _wait_old():
                for c in out_copies(i - 3, (j - 3) % _NBUF):
                    c.wait()

            if static:
                if i >= 3:
                    _wait_old()
            else:
                pl.when(i >= 3)(_wait_old)

            fire_gathers(i, j % _NIDX, j % _NBUF)
            transpose_and_write(i - 1, (j - 1) % _NIDX, (j - 1) % _NBUF)

            def _prefetch():
                idx_copy(i + 2, (j + 2) % _NIDX).start()

            if static:
                if i + 2 < n_chunks:
                    _prefetch()
            else:
                pl.when(i + 2 < n_chunks)(_prefetch)

        idx_copy(0, 0).start()
        idx_copy(1, 1).start()
        idx_copy(0, 0).wait()
        fire_gathers(0, 0, 0)
        idx_copy(2, 2).start()

        assert n_chunks % 4 == 0 and n_chunks >= 8

        def quad_body(p, carry):
            for j in range(4):
                step(p * 4 + j + 1, j + 1)
            return carry

        lax.fori_loop(0, (n_chunks - 4) // 4, quad_body, 0)

        for i in range(n_chunks - 3, n_chunks):
            step(i, i)
        for c in out_copies(n_chunks - 3, (n_chunks - 3) % _NBUF):
            c.wait()
        transpose_and_write(n_chunks - 1,
                            (n_chunks - 1) % _NIDX, (n_chunks - 1) % _NBUF)
        for c in out_copies(n_chunks - 2, (n_chunks - 2) % _NBUF):
            c.wait()
        for c in out_copies(n_chunks - 1, (n_chunks - 1) % _NBUF):
            c.wait()

    out2 = emb(xt2, table)
    out = (out2.reshape(S, dts, bblocks, 8, _GW)
           .transpose(0, 1, 3, 2, 4)
           .reshape(S, D, B0)
           .transpose(2, 0, 1))
    return out
